# Initial kernel scaffold; baseline (speedup 1.0000x reference)
import jax
import jax.numpy as jnp
from jax.experimental import pallas as pl


def kernel(species, features, radial_aev, atom_index12, params):
    raise NotImplementedError("write your pallas kernel here")


# speedup vs baseline: 9.2385x; 9.2385x over previous
import functools

import jax
import jax.numpy as jnp
from jax import lax
from jax.experimental import pallas as pl
from jax.experimental.pallas import tpu as pltpu
from jax.experimental.pallas import tpu_sc as plsc

N_ATOMS = 10000
F = 128
R = 64
N_PAIRS = 320000
NC = 2
NS = 16
CH = 80
PAIRS_PER_TILE = N_PAIRS // (NC * NS)
N_PAD = 10240
ROWS_PER_TILE = N_PAD // NS


def _sc_scatter_body(radial_hbm, idx0_hbm, idx1_hbm, out_hbm,
                     idx0_v, idx1_v, rows_v, acc):
    c = lax.axis_index("c")
    s = lax.axis_index("s")
    wid = c * NS + s

    z16 = jnp.zeros((16,), jnp.float32)
    for i in range(CH):
        for k in range(R // 16):
            rows_v[i, pl.ds(k * 16, 16)] = z16
    row0 = s * ROWS_PER_TILE
    for j in range(ROWS_PER_TILE // CH):
        pltpu.sync_copy(rows_v, acc.at[pl.ds(row0 + j * CH, CH)])
    plsc.subcore_barrier()

    pair_base = wid * PAIRS_PER_TILE

    def chunk(i, carry):
        pb = pair_base + i * CH
        pltpu.sync_copy(idx0_hbm.at[pl.ds(pb, CH)], idx0_v)
        pltpu.sync_copy(idx1_hbm.at[pl.ds(pb, CH)], idx1_v)
        pltpu.sync_copy(radial_hbm.at[pl.ds(pb, CH)], rows_v)
        pltpu.sync_copy(rows_v, acc.at[idx0_v], add=True)
        pltpu.sync_copy(rows_v, acc.at[idx1_v], add=True)
        return carry

    lax.fori_loop(0, PAIRS_PER_TILE // CH, chunk, 0)
    plsc.subcore_barrier()

    for j in range(ROWS_PER_TILE // CH):
        pltpu.sync_copy(acc.at[pl.ds(row0 + j * CH, CH)], rows_v)
        pltpu.sync_copy(rows_v, out_hbm.at[c, pl.ds(row0 + j * CH, CH)])


@jax.jit
def _sc_scatter(radial_aev, idx0, idx1):
    mesh = plsc.VectorSubcoreMesh(core_axis_name="c", subcore_axis_name="s")
    return pl.kernel(
        _sc_scatter_body,
        out_type=jax.ShapeDtypeStruct((NC, N_PAD, R), jnp.float32),
        mesh=mesh,
        compiler_params=pltpu.CompilerParams(use_tc_tiling_on_sc=False),
        scratch_types=[
            pltpu.VMEM((CH,), jnp.int32),
            pltpu.VMEM((CH,), jnp.int32),
            pltpu.VMEM((CH, R), jnp.float32),
            pltpu.VMEM_SHARED((N_PAD, R), jnp.float32),
        ],
    )(radial_aev, idx0, idx1)


_LOG2 = 0.6931471805599453


def _sp(x):
    return jnp.maximum(x, 0.0) + jnp.log(1.0 + jnp.exp(-jnp.abs(x))) - _LOG2


def _mm(x, w):
    return lax.dot_general(x, w, (((1,), (1,)), ((), ())),
                           preferred_element_type=jnp.float32)


def _dense_body(species_ref, feat_ref, part_ref, *rest):
    wrefs = rest[:-2]
    energy_ref, outfeat_ref = rest[-2:]
    w = [r[...] for r in wrefs]
    (WI, bI, WJ, bJ, Wg,
     i10, i11, i12, i13, i20, i21, i22, i23, i30, i31, i32, i33,
     Wint, bint, gate,
     a10, a11, a12, a13, a20, a21, a22, a23,
     o10, o11, o12, o13,
     Wout, bout) = w

    def res_block(x, W1, b1, W2, b2):
        out = _mm(_sp(x), W1) + b1
        return _mm(_sp(out), W2) + b2 + x

    x = feat_ref[...]
    mask = species_ref[...] != -1
    af = _sp(x)
    g = _sp(_mm(af, WJ) + bJ)
    protoI = _sp(_mm(af, WI) + bI)
    A = part_ref[0] + part_ref[1]
    S = _mm(A, Wg)
    proto = S * g + jnp.where(mask, protoI, 0.0)
    msg = res_block(proto, i10, i11, i12, i13)
    msg = res_block(msg, i20, i21, i22, i23)
    msg = res_block(msg, i30, i31, i32, i33)
    dense = x * gate + _mm(_sp(msg), Wint) + bint
    dense = res_block(dense, a10, a11, a12, a13)
    dense = res_block(dense, a20, a21, a22, a23)
    t = res_block(dense, o10, o11, o12, o13)
    e = jnp.sum(_sp(t) * Wout + bout, axis=1, keepdims=True)
    energy_ref[...] = jnp.where(mask, e, 0.0)
    outfeat_ref[...] = jnp.where(mask, dense, 0.0)


def _dense_chain(species_flat, features, partial, weights, block_rows=2000):
    grid = (N_ATOMS // block_rows,)
    row_spec = lambda cols: pl.BlockSpec((block_rows, cols), lambda i: (i, 0))
    in_specs = [
        row_spec(1),
        row_spec(F),
        pl.BlockSpec((NC, block_rows, R), lambda i: (0, i, 0)),
    ] + [pl.BlockSpec(wi.shape, lambda i: (0, 0)) for wi in weights]
    out_specs = [row_spec(1), row_spec(F)]
    energies, out_features = pl.pallas_call(
        _dense_body,
        grid=grid,
        in_specs=in_specs,
        out_specs=out_specs,
        out_shape=[
            jax.ShapeDtypeStruct((N_ATOMS, 1), jnp.float32),
            jax.ShapeDtypeStruct((N_ATOMS, F), jnp.float32),
        ],
    )(species_flat, features, partial, *weights)
    return energies, out_features


def kernel(species, features, radial_aev, atom_index12, params):
    idx = atom_index12.astype(jnp.int32)
    partial = _sc_scatter(radial_aev, idx[0], idx[1])

    def lin2(p):
        W, b = p
        return [W, b.reshape(1, F)]

    pr = params
    weights = (
        lin2(pr['linearI']) + lin2(pr['linearJ']) + [pr['gating_linear_W']]
        + [t for blk in pr['inter_res'] for p in blk for t in lin2(p)]
        + lin2(pr['interaction_linear'])
        + [pr['gating_vector'].reshape(1, F)]
        + [t for blk in pr['atomic_res'] for p in blk for t in lin2(p)]
        + [t for blk in pr['output_res'] for p in blk for t in lin2(p)]
        + [pr['output_linear'][0],
           jnp.broadcast_to(pr['output_linear'][1].reshape(1, 1) / F, (1, F))]
    )
    species_flat = species.reshape(-1, 1).astype(jnp.int32)
    energies, out_features = _dense_chain(species_flat, features, partial, weights)
    return energies.reshape(species.shape[0], species.shape[1]), out_features


# speedup vs baseline: 14.7095x; 1.5922x over previous
import functools

import jax
import jax.numpy as jnp
from jax import lax
from jax.experimental import pallas as pl
from jax.experimental.pallas import tpu as pltpu
from jax.experimental.pallas import tpu_sc as plsc

N_ATOMS = 10000
F = 128
R = 64
N_PAIRS = 320000
NC = 2
NS = 16
NW = NC * NS
PAIRS_PER_TILE = N_PAIRS // NW
CHUNK = 200
SUB = 100
NSUB = CHUNK // SUB
ITERS = PAIRS_PER_TILE // CHUNK
NBUF = 2
N_PAD = 10240
ROWS_PER_TILE = N_PAD // NS
ZH = 80


def _sc_scatter_body(radial_hbm, idx0_hbm, idx1_hbm, out_hbm,
                     idx_v, rows_v, sems, acc):
    c = lax.axis_index("c")
    s = lax.axis_index("s")
    wid = c * NS + s

    z16 = jnp.zeros((16,), jnp.float32)
    for i in range(ZH):
        for k in range(R // 16):
            rows_v[0, i, pl.ds(k * 16, 16)] = z16
    row0 = s * ROWS_PER_TILE
    for j in range(ROWS_PER_TILE // ZH):
        pltpu.sync_copy(rows_v.at[0, pl.ds(0, ZH)], acc.at[pl.ds(row0 + j * ZH, ZH)])
    plsc.subcore_barrier()

    pair0 = wid * PAIRS_PER_TILE

    def issue_load(j, b):
        pltpu.async_copy(idx0_hbm.at[wid, j], idx_v.at[b, 0], sems.at[b])
        pltpu.async_copy(idx1_hbm.at[wid, j], idx_v.at[b, 1], sems.at[b])
        pltpu.async_copy(radial_hbm.at[pl.ds(pair0 + j * CHUNK, CHUNK)],
                         rows_v.at[b], sems.at[b])

    def wait_load(j, b):
        pltpu.make_async_copy(idx0_hbm.at[wid, j], idx_v.at[b, 0], sems.at[b]).wait()
        pltpu.make_async_copy(idx1_hbm.at[wid, j], idx_v.at[b, 1], sems.at[b]).wait()
        pltpu.make_async_copy(radial_hbm.at[pl.ds(pair0 + j * CHUNK, CHUNK)],
                              rows_v.at[b], sems.at[b]).wait()

    for b in range(NBUF):
        issue_load(b, b)

    @pl.loop(0, ITERS, step=NBUF)
    def _pipeline(g):
        for b in range(NBUF):
            j = g + b
            wait_load(j, b)
            for k in range(NSUB):
                src = rows_v.at[b, pl.ds(k * SUB, SUB)]
                pltpu.sync_copy(src, acc.at[idx_v.at[b, 0, k]], add=True)
                pltpu.sync_copy(src, acc.at[idx_v.at[b, 1, k]], add=True)
            nj = j + NBUF

            @pl.when(nj < ITERS)
            def _():
                issue_load(nj, b)

    plsc.subcore_barrier()

    for j in range(ROWS_PER_TILE // ZH):
        pltpu.sync_copy(acc.at[pl.ds(row0 + j * ZH, ZH)], rows_v.at[0, pl.ds(0, ZH)])
        pltpu.sync_copy(rows_v.at[0, pl.ds(0, ZH)], out_hbm.at[c, pl.ds(row0 + j * ZH, ZH)])


@jax.jit
def _sc_scatter(radial_aev, idx0, idx1):
    mesh = plsc.VectorSubcoreMesh(core_axis_name="c", subcore_axis_name="s")
    return pl.kernel(
        _sc_scatter_body,
        out_type=jax.ShapeDtypeStruct((NC, N_PAD, R), jnp.float32),
        mesh=mesh,
        compiler_params=pltpu.CompilerParams(use_tc_tiling_on_sc=False),
        scratch_types=[
            pltpu.VMEM((NBUF, 2, NSUB, SUB), jnp.int32),
            pltpu.VMEM((NBUF, CHUNK, R), jnp.float32),
            pltpu.SemaphoreType.DMA((NBUF,)),
            pltpu.VMEM_SHARED((N_PAD, R), jnp.float32),
        ],
    )(radial_aev,
      idx0.reshape(NW, ITERS, NSUB, SUB),
      idx1.reshape(NW, ITERS, NSUB, SUB))


_LOG2 = 0.6931471805599453


def _sp(x):
    return jnp.maximum(x, 0.0) + jnp.log(1.0 + jnp.exp(-jnp.abs(x))) - _LOG2


def _mm(x, w):
    return lax.dot_general(x, w, (((1,), (1,)), ((), ())),
                           preferred_element_type=jnp.float32)


def _dense_body(species_ref, feat_ref, part_ref, *rest):
    wrefs = rest[:-2]
    energy_ref, outfeat_ref = rest[-2:]
    w = [r[...] for r in wrefs]
    (WI, bI, WJ, bJ, Wg,
     i10, i11, i12, i13, i20, i21, i22, i23, i30, i31, i32, i33,
     Wint, bint, gate,
     a10, a11, a12, a13, a20, a21, a22, a23,
     o10, o11, o12, o13,
     Wout, bout) = w

    def res_block(x, W1, b1, W2, b2):
        out = _mm(_sp(x), W1) + b1
        return _mm(_sp(out), W2) + b2 + x

    x = feat_ref[...]
    mask = species_ref[...] != -1
    af = _sp(x)
    g = _sp(_mm(af, WJ) + bJ)
    protoI = _sp(_mm(af, WI) + bI)
    A = part_ref[0] + part_ref[1]
    S = _mm(A, Wg)
    proto = S * g + jnp.where(mask, protoI, 0.0)
    msg = res_block(proto, i10, i11, i12, i13)
    msg = res_block(msg, i20, i21, i22, i23)
    msg = res_block(msg, i30, i31, i32, i33)
    dense = x * gate + _mm(_sp(msg), Wint) + bint
    dense = res_block(dense, a10, a11, a12, a13)
    dense = res_block(dense, a20, a21, a22, a23)
    t = res_block(dense, o10, o11, o12, o13)
    e = jnp.sum(_sp(t) * Wout + bout, axis=1, keepdims=True)
    energy_ref[...] = jnp.where(mask, e, 0.0)
    outfeat_ref[...] = jnp.where(mask, dense, 0.0)


def _dense_chain(species_flat, features, partial, weights, block_rows=2000):
    grid = (N_ATOMS // block_rows,)
    row_spec = lambda cols: pl.BlockSpec((block_rows, cols), lambda i: (i, 0))
    in_specs = [
        row_spec(1),
        row_spec(F),
        pl.BlockSpec((NC, block_rows, R), lambda i: (0, i, 0)),
    ] + [pl.BlockSpec(wi.shape, lambda i: (0, 0)) for wi in weights]
    out_specs = [row_spec(1), row_spec(F)]
    energies, out_features = pl.pallas_call(
        _dense_body,
        grid=grid,
        in_specs=in_specs,
        out_specs=out_specs,
        out_shape=[
            jax.ShapeDtypeStruct((N_ATOMS, 1), jnp.float32),
            jax.ShapeDtypeStruct((N_ATOMS, F), jnp.float32),
        ],
    )(species_flat, features, partial, *weights)
    return energies, out_features


def kernel(species, features, radial_aev, atom_index12, params):
    idx = atom_index12.astype(jnp.int32)
    partial = _sc_scatter(radial_aev, idx[0], idx[1])

    def lin2(p):
        W, b = p
        return [W, b.reshape(1, F)]

    pr = params
    weights = (
        lin2(pr['linearI']) + lin2(pr['linearJ']) + [pr['gating_linear_W']]
        + [t for blk in pr['inter_res'] for p in blk for t in lin2(p)]
        + lin2(pr['interaction_linear'])
        + [pr['gating_vector'].reshape(1, F)]
        + [t for blk in pr['atomic_res'] for p in blk for t in lin2(p)]
        + [t for blk in pr['output_res'] for p in blk for t in lin2(p)]
        + [pr['output_linear'][0],
           jnp.broadcast_to(pr['output_linear'][1].reshape(1, 1) / F, (1, F))]
    )
    species_flat = species.reshape(-1, 1).astype(jnp.int32)
    energies, out_features = _dense_chain(species_flat, features, partial, weights)
    return energies.reshape(species.shape[0], species.shape[1]), out_features
